# baseline (speedup 1.0000x reference)
import functools

import jax
import jax.numpy as jnp
from jax import lax
from jax.experimental import pallas as pl
from jax.experimental.pallas import tpu as pltpu
from jax.experimental.pallas import tpu_sc as plsc

_N = 10000
_E = 320000
_D = 128
_NC = 2
_NS = 16
_NW = _NC * _NS
_L = 16

_CHUNK = 128
_ITERS = 80
_EPAD = _NW * _ITERS * _CHUNK
_STRIPE = 624


def _mm_body(x_ref, w_ref, o_ref):
    o_ref[...] = jnp.dot(x_ref[...], w_ref[...],
                         preferred_element_type=jnp.float32)


def _matmul(x, w):
    return pl.pallas_call(
        _mm_body,
        grid=(5,),
        in_specs=[
            pl.BlockSpec((2000, _D), lambda i: (i, 0)),
            pl.BlockSpec((_D, _D), lambda i: (0, 0)),
        ],
        out_specs=pl.BlockSpec((2000, _D), lambda i: (i, 0)),
        out_shape=jax.ShapeDtypeStruct((_N, _D), jnp.float32),
    )(x, w)


_mesh = plsc.VectorSubcoreMesh(core_axis_name="c", subcore_axis_name="s")


@functools.partial(
    pl.kernel,
    out_type=jax.ShapeDtypeStruct((_NC, _N, _D), jnp.float32),
    mesh=_mesh,
    scratch_types=[
        pltpu.VMEM((_CHUNK,), jnp.int32),
        pltpu.VMEM((_CHUNK,), jnp.int32),
        pltpu.VMEM((_CHUNK,), jnp.float32),
        pltpu.VMEM((_CHUNK, _D), jnp.float32),
        pltpu.VMEM_SHARED((_N, _D), jnp.float32),
        pltpu.SemaphoreType.DMA,
    ],
)
def _spmm(src_hbm, dst_hbm, ew_hbm, sup_hbm, out_hbm,
          src_v, dst_v, w_v, rows_v, acc, gat):
    c = lax.axis_index("c")
    s = lax.axis_index("s")
    wid = s * _NC + c

    def _zero_row(i, carry):
        for j in range(_D // _L):
            rows_v[i, pl.ds(j * _L, _L)] = jnp.zeros((_L,), jnp.float32)
        return carry
    lax.fori_loop(0, _CHUNK, _zero_row, 0)

    stripe = s * _STRIPE
    for off in (0, 128, 256, 384, 496):
        pltpu.sync_copy(rows_v, acc.at[pl.ds(stripe + off, _CHUNK)])

    @pl.when(s == _NS - 1)
    def _zero_tail():
        pltpu.sync_copy(rows_v, acc.at[pl.ds(_N - _CHUNK, _CHUNK)])
    plsc.subcore_barrier()

    def _body(it, carry):
        base = (it * _NW + wid) * _CHUNK
        pltpu.sync_copy(src_hbm.at[pl.ds(base, _CHUNK)], src_v)
        pltpu.sync_copy(dst_hbm.at[pl.ds(base, _CHUNK)], dst_v)
        pltpu.sync_copy(ew_hbm.at[pl.ds(base, _CHUNK)], w_v)
        pltpu.async_copy(sup_hbm.at[src_v], rows_v, gat).wait()

        def _scale16(g, carry2):
            wvec = w_v[pl.ds(g * _L, _L)]
            for l in range(_L):
                wl = wvec.at[jnp.full((_L,), l, jnp.int32)].get(
                    mode="promise_in_bounds")
                r = g * _L + l
                for j in range(_D // _L):
                    sl = pl.ds(j * _L, _L)
                    rows_v[r, sl] = rows_v[r, sl] * wl
            return carry2
        lax.fori_loop(0, _CHUNK // _L, _scale16, 0)

        pltpu.sync_copy(rows_v, acc.at[dst_v], add=True)
        return carry
    lax.fori_loop(0, _ITERS, _body, 0)

    plsc.subcore_barrier()
    for off in (0, 128, 256, 384, 496):
        pltpu.sync_copy(acc.at[pl.ds(stripe + off, _CHUNK)],
                        out_hbm.at[c, pl.ds(stripe + off, _CHUNK)])

    @pl.when(s == _NS - 1)
    def _write_tail():
        pltpu.sync_copy(acc.at[pl.ds(_N - _CHUNK, _CHUNK)],
                        out_hbm.at[c, pl.ds(_N - _CHUNK, _CHUNK)])


def _comb_body(p_ref, b_ref, o_ref):
    o_ref[...] = p_ref[0] + p_ref[1] + b_ref[...]


def _combine(partials, bias2d):
    return pl.pallas_call(
        _comb_body,
        grid=(5,),
        in_specs=[
            pl.BlockSpec((_NC, 2000, _D), lambda i: (0, i, 0)),
            pl.BlockSpec((1, _D), lambda i: (0, 0)),
        ],
        out_specs=pl.BlockSpec((2000, _D), lambda i: (i, 0)),
        out_shape=jax.ShapeDtypeStruct((_N, _D), jnp.float32),
    )(partials, bias2d)


def kernel(x, edge_index, edge_weight, weight, bias):
    support = _matmul(x, weight)
    pad = _EPAD - _E
    ei = jnp.pad(edge_index, ((0, 0), (0, pad)))
    ew = jnp.pad(edge_weight, (0, pad))
    partials = _spmm(ei[0], ei[1], ew, support)
    return _combine(partials, bias.reshape(1, _D))


# speedup vs baseline: 2.1255x; 2.1255x over previous
import functools

import jax
import jax.numpy as jnp
from jax import lax
from jax.experimental import pallas as pl
from jax.experimental.pallas import tpu as pltpu
from jax.experimental.pallas import tpu_sc as plsc

_N = 10000
_E = 320000
_D = 128
_NC = 2
_NS = 16
_NW = _NC * _NS
_L = 16

_CHUNK = 128
_NCHUNKS = _E // _CHUNK
_BASE = _NCHUNKS // _NW
_EXTRA = _NCHUNKS - _BASE * _NW
_STRIPE = 624


def _mm_body(x_ref, w_ref, o_ref):
    o_ref[...] = jnp.dot(x_ref[...], w_ref[...],
                         preferred_element_type=jnp.float32)


def _matmul(x, w):
    return pl.pallas_call(
        _mm_body,
        grid=(5,),
        in_specs=[
            pl.BlockSpec((2000, _D), lambda i: (i, 0)),
            pl.BlockSpec((_D, _D), lambda i: (0, 0)),
        ],
        out_specs=pl.BlockSpec((2000, _D), lambda i: (i, 0)),
        out_shape=jax.ShapeDtypeStruct((_N, _D), jnp.float32),
    )(x, w)


_mesh = plsc.VectorSubcoreMesh(core_axis_name="c", subcore_axis_name="s")


@functools.partial(
    pl.kernel,
    out_type=jax.ShapeDtypeStruct((_NC, _N, _D), jnp.float32),
    mesh=_mesh,
    compiler_params=pltpu.CompilerParams(needs_layout_passes=False),
    scratch_types=[
        pltpu.VMEM((3, _CHUNK), jnp.int32),
        pltpu.VMEM((_CHUNK, _D), jnp.float32),
        pltpu.VMEM_SHARED((_N, _D), jnp.float32),
        pltpu.SemaphoreType.DMA,
    ],
)
def _spmm(eidx_hbm, sup_hbm, out_hbm, slab_v, rows_v, acc, gat):
    c = lax.axis_index("c")
    s = lax.axis_index("s")
    wid = s * _NC + c

    def _zero_row(i, carry):
        for j in range(_D // _L):
            rows_v[i, pl.ds(j * _L, _L)] = jnp.zeros((_L,), jnp.float32)
        return carry
    lax.fori_loop(0, _CHUNK, _zero_row, 0)

    stripe = s * _STRIPE
    for off in (0, 128, 256, 384, 496):
        pltpu.sync_copy(rows_v, acc.at[pl.ds(stripe + off, _CHUNK)])

    @pl.when(s == _NS - 1)
    def _zero_tail():
        pltpu.sync_copy(rows_v, acc.at[pl.ds(_N - _CHUNK, _CHUNK)])
    plsc.subcore_barrier()

    def _body(it, carry):
        pltpu.sync_copy(eidx_hbm.at[it * _NW + wid], slab_v)
        pltpu.async_copy(sup_hbm.at[slab_v.at[0]], rows_v, gat).wait()

        def _scale16(g, carry2):
            wvec = plsc.bitcast(slab_v[2, pl.ds(g * _L, _L)], jnp.float32)
            for l in range(_L):
                wl = wvec.at[jnp.full((_L,), l, jnp.int32)].get(
                    mode="promise_in_bounds")
                r = g * _L + l
                for j in range(_D // _L):
                    sl = pl.ds(j * _L, _L)
                    rows_v[r, sl] = rows_v[r, sl] * wl
            return carry2
        lax.fori_loop(0, _CHUNK // _L, _scale16, 0)

        pltpu.sync_copy(rows_v, acc.at[slab_v.at[1]], add=True)
        return carry
    nchunks = _BASE + jnp.where(wid < _EXTRA, 1, 0)
    lax.fori_loop(0, nchunks, _body, 0)

    plsc.subcore_barrier()
    for off in (0, 128, 256, 384, 496):
        pltpu.sync_copy(acc.at[pl.ds(stripe + off, _CHUNK)],
                        out_hbm.at[c, pl.ds(stripe + off, _CHUNK)])

    @pl.when(s == _NS - 1)
    def _write_tail():
        pltpu.sync_copy(acc.at[pl.ds(_N - _CHUNK, _CHUNK)],
                        out_hbm.at[c, pl.ds(_N - _CHUNK, _CHUNK)])


def _comb_body(p_ref, b_ref, o_ref):
    o_ref[...] = p_ref[0] + p_ref[1] + b_ref[...]


def _combine(partials, bias2d):
    return pl.pallas_call(
        _comb_body,
        grid=(5,),
        in_specs=[
            pl.BlockSpec((_NC, 2000, _D), lambda i: (0, i, 0)),
            pl.BlockSpec((1, _D), lambda i: (0, 0)),
        ],
        out_specs=pl.BlockSpec((2000, _D), lambda i: (i, 0)),
        out_shape=jax.ShapeDtypeStruct((_N, _D), jnp.float32),
    )(partials, bias2d)


def kernel(x, edge_index, edge_weight, weight, bias):
    support = _matmul(x, weight)
    ew_bits = jax.lax.bitcast_convert_type(edge_weight, jnp.int32)
    eidx3 = jnp.stack(
        [edge_index[0].reshape(_NCHUNKS, _CHUNK),
         edge_index[1].reshape(_NCHUNKS, _CHUNK),
         ew_bits.reshape(_NCHUNKS, _CHUNK)], axis=1)
    partials = _spmm(eidx3, support)
    return _combine(partials, bias.reshape(1, _D))


# speedup vs baseline: 3.7561x; 1.7671x over previous
import functools

import jax
import jax.numpy as jnp
from jax import lax
from jax.experimental import pallas as pl
from jax.experimental.pallas import tpu as pltpu
from jax.experimental.pallas import tpu_sc as plsc

_N = 10000
_E = 320000
_D = 128
_NC = 2
_NS = 16
_NW = _NC * _NS
_L = 16

_CHUNK = 128
_NCHUNKS = _E // _CHUNK
_BASE = _NCHUNKS // _NW
_EXTRA = _NCHUNKS - _BASE * _NW
_STRIPE = 624


def _mm_body(x_ref, w_ref, o_ref):
    o_ref[...] = jnp.dot(x_ref[...], w_ref[...],
                         preferred_element_type=jnp.float32)


def _matmul(x, w):
    return pl.pallas_call(
        _mm_body,
        grid=(5,),
        in_specs=[
            pl.BlockSpec((2000, _D), lambda i: (i, 0)),
            pl.BlockSpec((_D, _D), lambda i: (0, 0)),
        ],
        out_specs=pl.BlockSpec((2000, _D), lambda i: (i, 0)),
        out_shape=jax.ShapeDtypeStruct((_N, _D), jnp.float32),
    )(x, w)


_mesh = plsc.VectorSubcoreMesh(core_axis_name="c", subcore_axis_name="s")


@functools.partial(
    pl.kernel,
    out_type=jax.ShapeDtypeStruct((_NC, _N, _D), jnp.float32),
    mesh=_mesh,
    compiler_params=pltpu.CompilerParams(needs_layout_passes=False),
    scratch_types=[
        pltpu.VMEM((3, _CHUNK), jnp.int32),
        pltpu.VMEM((3, _CHUNK), jnp.int32),
        pltpu.VMEM((3, _CHUNK), jnp.int32),
        pltpu.VMEM((3, _CHUNK), jnp.int32),
        pltpu.VMEM((_CHUNK, _D), jnp.float32),
        pltpu.VMEM((_CHUNK, _D), jnp.float32),
        pltpu.VMEM_SHARED((_N, _D), jnp.float32),
        pltpu.SemaphoreType.DMA,
        pltpu.SemaphoreType.DMA,
        pltpu.SemaphoreType.DMA,
        pltpu.SemaphoreType.DMA,
        pltpu.SemaphoreType.DMA,
        pltpu.SemaphoreType.DMA,
        pltpu.SemaphoreType.DMA,
        pltpu.SemaphoreType.DMA,
    ],
)
def _spmm(eidx_hbm, sup_hbm, out_hbm, sl0, sl1, sl2, sl3,
          rowsA, rowsB, acc, sm0, sm1, sm2, sm3, gA, gB, scA, scB):
    c = lax.axis_index("c")
    s = lax.axis_index("s")
    wid = s * _NC + c
    slabs = (sl0, sl1, sl2, sl3)
    slsems = (sm0, sm1, sm2, sm3)
    rows = (rowsA, rowsB)
    gsems = (gA, gB)
    scsems = (scA, scB)

    def _chunk_of(it):
        return it * _NW + wid

    def _start_slab(it, kslot):
        pltpu.async_copy(eidx_hbm.at[_chunk_of(it)], slabs[kslot],
                         slsems[kslot])

    def _wait_slab(it, kslot):
        pltpu.make_async_copy(eidx_hbm.at[_chunk_of(it)], slabs[kslot],
                              slsems[kslot]).wait()

    def _start_gather(kslot, kbuf):
        pltpu.async_copy(sup_hbm.at[slabs[kslot].at[0]], rows[kbuf],
                         gsems[kbuf])

    def _wait_gather(kslot, kbuf):
        pltpu.make_async_copy(sup_hbm.at[slabs[kslot].at[0]], rows[kbuf],
                              gsems[kbuf]).wait()

    def _start_scatter(kslot, kbuf):
        pltpu.async_copy(rows[kbuf], acc.at[slabs[kslot].at[1]],
                         scsems[kbuf], add=True)

    def _wait_scatter(kslot, kbuf):
        pltpu.make_async_copy(rows[kbuf], acc.at[slabs[kslot].at[1]],
                              scsems[kbuf]).wait()

    def _scale(kslot, kbuf):
        rb = rows[kbuf]
        wref = slabs[kslot]

        def _scale16(g, carry2):
            wvec = plsc.bitcast(wref[2, pl.ds(g * _L, _L)], jnp.float32)
            for l in range(_L):
                wl = wvec.at[jnp.full((_L,), l, jnp.int32)].get(
                    mode="promise_in_bounds")
                r = g * _L + l
                for j in range(_D // _L):
                    slc = pl.ds(j * _L, _L)
                    rb[r, slc] = rb[r, slc] * wl
            return carry2
        lax.fori_loop(0, _CHUNK // _L, _scale16, 0)

    def _zero_row(i, carry):
        for j in range(_D // _L):
            rowsA[i, pl.ds(j * _L, _L)] = jnp.zeros((_L,), jnp.float32)
        return carry
    lax.fori_loop(0, _CHUNK, _zero_row, 0)

    stripe = s * _STRIPE
    for off in (0, 128, 256, 384, 496):
        pltpu.sync_copy(rowsA, acc.at[pl.ds(stripe + off, _CHUNK)])

    @pl.when(s == _NS - 1)
    def _zero_tail():
        pltpu.sync_copy(rowsA, acc.at[pl.ds(_N - _CHUNK, _CHUNK)])
    plsc.subcore_barrier()

    nchunks = _BASE + jnp.where(wid < _EXTRA, 1, 0)

    _start_slab(0, 0)
    _wait_slab(0, 0)
    _start_gather(0, 0)
    _start_slab(1, 1)

    def _sub(it, k):
        @pl.when(it + 1 < nchunks)
        def _():
            _wait_slab(it + 1, (k + 1) & 3)

            @pl.when(it >= 1)
            def _():
                _wait_scatter((k + 3) & 3, (k + 1) & 1)
            _start_gather((k + 1) & 3, (k + 1) & 1)

        @pl.when(it + 2 < nchunks)
        def _():
            _start_slab(it + 2, (k + 2) & 3)
        _wait_gather(k & 3, k & 1)
        _scale(k & 3, k & 1)
        _start_scatter(k & 3, k & 1)

    def _body(t, carry):
        base_it = t * 4
        for k in range(4):
            @pl.when(base_it + k < nchunks)
            def _():
                _sub(base_it + k, k)
        return carry
    lax.fori_loop(0, (nchunks + 3) // 4, _body, 0)

    _wait_scatter(0, 0)
    _wait_scatter(1, 1)

    plsc.subcore_barrier()
    for off in (0, 128, 256, 384, 496):
        pltpu.sync_copy(acc.at[pl.ds(stripe + off, _CHUNK)],
                        out_hbm.at[c, pl.ds(stripe + off, _CHUNK)])

    @pl.when(s == _NS - 1)
    def _write_tail():
        pltpu.sync_copy(acc.at[pl.ds(_N - _CHUNK, _CHUNK)],
                        out_hbm.at[c, pl.ds(_N - _CHUNK, _CHUNK)])


def _comb_body(p_ref, b_ref, o_ref):
    o_ref[...] = p_ref[0] + p_ref[1] + b_ref[...]


def _combine(partials, bias2d):
    return pl.pallas_call(
        _comb_body,
        grid=(5,),
        in_specs=[
            pl.BlockSpec((_NC, 2000, _D), lambda i: (0, i, 0)),
            pl.BlockSpec((1, _D), lambda i: (0, 0)),
        ],
        out_specs=pl.BlockSpec((2000, _D), lambda i: (i, 0)),
        out_shape=jax.ShapeDtypeStruct((_N, _D), jnp.float32),
    )(partials, bias2d)


def kernel(x, edge_index, edge_weight, weight, bias):
    support = _matmul(x, weight)
    ew_bits = jax.lax.bitcast_convert_type(edge_weight, jnp.int32)
    eidx3 = jnp.stack(
        [edge_index[0].reshape(_NCHUNKS, _CHUNK),
         edge_index[1].reshape(_NCHUNKS, _CHUNK),
         ew_bits.reshape(_NCHUNKS, _CHUNK)], axis=1)
    partials = _spmm(eidx3, support)
    return _combine(partials, bias.reshape(1, _D))


# speedup vs baseline: 3.7589x; 1.0007x over previous
import functools

import jax
import jax.numpy as jnp
from jax import lax
from jax.experimental import pallas as pl
from jax.experimental.pallas import tpu as pltpu
from jax.experimental.pallas import tpu_sc as plsc

_N = 10000
_E = 320000
_D = 128
_NC = 2
_NS = 16
_NW = _NC * _NS
_L = 16

_CHUNK = 128
_NCHUNKS = _E // _CHUNK
_BASE = _NCHUNKS // _NW
_EXTRA = _NCHUNKS - _BASE * _NW
_STRIPE = 624


def _mm_body(x_ref, w_ref, o_ref):
    o_ref[...] = jnp.dot(x_ref[...], w_ref[...],
                         preferred_element_type=jnp.float32)


def _matmul(x, w):
    return pl.pallas_call(
        _mm_body,
        grid=(5,),
        in_specs=[
            pl.BlockSpec((2000, _D), lambda i: (i, 0)),
            pl.BlockSpec((_D, _D), lambda i: (0, 0)),
        ],
        out_specs=pl.BlockSpec((2000, _D), lambda i: (i, 0)),
        out_shape=jax.ShapeDtypeStruct((_N, _D), jnp.float32),
    )(x, w)


_mesh = plsc.VectorSubcoreMesh(core_axis_name="c", subcore_axis_name="s")


@functools.partial(
    pl.kernel,
    out_type=jax.ShapeDtypeStruct((_NC, _N, _D), jnp.float32),
    mesh=_mesh,
    compiler_params=pltpu.CompilerParams(needs_layout_passes=False),
    scratch_types=[
        pltpu.VMEM((3, _CHUNK), jnp.int32),
        pltpu.VMEM((3, _CHUNK), jnp.int32),
        pltpu.VMEM((3, _CHUNK), jnp.int32),
        pltpu.VMEM((3, _CHUNK), jnp.int32),
        pltpu.VMEM((_CHUNK, _D), jnp.float32),
        pltpu.VMEM((_CHUNK, _D), jnp.float32),
        pltpu.VMEM_SHARED((_N, _D), jnp.float32),
        pltpu.SemaphoreType.DMA,
        pltpu.SemaphoreType.DMA,
        pltpu.SemaphoreType.DMA,
        pltpu.SemaphoreType.DMA,
        pltpu.SemaphoreType.DMA,
        pltpu.SemaphoreType.DMA,
        pltpu.SemaphoreType.DMA,
        pltpu.SemaphoreType.DMA,
    ],
)
def _spmm(eidx_hbm, sup_hbm, out_hbm, sl0, sl1, sl2, sl3,
          rowsA, rowsB, acc, sm0, sm1, sm2, sm3, gA, gB, scA, scB):
    c = lax.axis_index("c")
    s = lax.axis_index("s")
    wid = s * _NC + c
    slabs = (sl0, sl1, sl2, sl3)
    slsems = (sm0, sm1, sm2, sm3)
    rows = (rowsA, rowsB)
    gsems = (gA, gB)
    scsems = (scA, scB)

    def _chunk_of(it):
        return it * _NW + wid

    def _start_slab(it, kslot):
        pltpu.async_copy(eidx_hbm.at[_chunk_of(it)], slabs[kslot],
                         slsems[kslot])

    def _wait_slab(it, kslot):
        pltpu.make_async_copy(eidx_hbm.at[_chunk_of(it)], slabs[kslot],
                              slsems[kslot]).wait()

    def _start_gather(kslot, kbuf):
        pltpu.async_copy(sup_hbm.at[slabs[kslot].at[0, pl.ds(0, 64)]],
                         rows[kbuf].at[pl.ds(0, 64)], gsems[kbuf])
        pltpu.async_copy(sup_hbm.at[slabs[kslot].at[0, pl.ds(64, 64)]],
                         rows[kbuf].at[pl.ds(64, 64)], gsems[kbuf])

    def _wait_gather(kslot, kbuf):
        pltpu.make_async_copy(sup_hbm.at[slabs[kslot].at[0, pl.ds(0, 64)]],
                              rows[kbuf].at[pl.ds(0, 64)],
                              gsems[kbuf]).wait()
        pltpu.make_async_copy(sup_hbm.at[slabs[kslot].at[0, pl.ds(64, 64)]],
                              rows[kbuf].at[pl.ds(64, 64)],
                              gsems[kbuf]).wait()

    def _start_scatter(kslot, kbuf):
        pltpu.async_copy(rows[kbuf], acc.at[slabs[kslot].at[1]],
                         scsems[kbuf], add=True)

    def _wait_scatter(kslot, kbuf):
        pltpu.make_async_copy(rows[kbuf], acc.at[slabs[kslot].at[1]],
                              scsems[kbuf]).wait()

    def _scale(kslot, kbuf):
        rb = rows[kbuf]
        wref = slabs[kslot]

        def _scale16(g, carry2):
            wvec = plsc.bitcast(wref[2, pl.ds(g * _L, _L)], jnp.float32)
            for l in range(_L):
                wl = wvec.at[jnp.full((_L,), l, jnp.int32)].get(
                    mode="promise_in_bounds")
                r = g * _L + l
                for j in range(_D // _L):
                    slc = pl.ds(j * _L, _L)
                    rb[r, slc] = rb[r, slc] * wl
            return carry2
        lax.fori_loop(0, _CHUNK // _L, _scale16, 0)

    def _zero_row(i, carry):
        for j in range(_D // _L):
            rowsA[i, pl.ds(j * _L, _L)] = jnp.zeros((_L,), jnp.float32)
        return carry
    lax.fori_loop(0, _CHUNK, _zero_row, 0)

    stripe = s * _STRIPE
    for off in (0, 128, 256, 384, 496):
        pltpu.sync_copy(rowsA, acc.at[pl.ds(stripe + off, _CHUNK)])

    @pl.when(s == _NS - 1)
    def _zero_tail():
        pltpu.sync_copy(rowsA, acc.at[pl.ds(_N - _CHUNK, _CHUNK)])
    plsc.subcore_barrier()

    nchunks = _BASE + jnp.where(wid < _EXTRA, 1, 0)

    _start_slab(0, 0)
    _wait_slab(0, 0)
    _start_gather(0, 0)
    _start_slab(1, 1)

    def _sub(it, k):
        @pl.when(it + 1 < nchunks)
        def _():
            _wait_slab(it + 1, (k + 1) & 3)

            @pl.when(it >= 1)
            def _():
                _wait_scatter((k + 3) & 3, (k + 1) & 1)
            _start_gather((k + 1) & 3, (k + 1) & 1)

        @pl.when(it + 2 < nchunks)
        def _():
            _start_slab(it + 2, (k + 2) & 3)
        _wait_gather(k & 3, k & 1)
        _scale(k & 3, k & 1)
        _start_scatter(k & 3, k & 1)

    def _body(t, carry):
        base_it = t * 4
        for k in range(4):
            @pl.when(base_it + k < nchunks)
            def _():
                _sub(base_it + k, k)
        return carry
    lax.fori_loop(0, (nchunks + 3) // 4, _body, 0)

    _wait_scatter(0, 0)
    _wait_scatter(1, 1)

    plsc.subcore_barrier()
    for off in (0, 128, 256, 384, 496):
        pltpu.sync_copy(acc.at[pl.ds(stripe + off, _CHUNK)],
                        out_hbm.at[c, pl.ds(stripe + off, _CHUNK)])

    @pl.when(s == _NS - 1)
    def _write_tail():
        pltpu.sync_copy(acc.at[pl.ds(_N - _CHUNK, _CHUNK)],
                        out_hbm.at[c, pl.ds(_N - _CHUNK, _CHUNK)])


def _comb_body(p_ref, b_ref, o_ref):
    o_ref[...] = p_ref[0] + p_ref[1] + b_ref[...]


def _combine(partials, bias2d):
    return pl.pallas_call(
        _comb_body,
        grid=(5,),
        in_specs=[
            pl.BlockSpec((_NC, 2000, _D), lambda i: (0, i, 0)),
            pl.BlockSpec((1, _D), lambda i: (0, 0)),
        ],
        out_specs=pl.BlockSpec((2000, _D), lambda i: (i, 0)),
        out_shape=jax.ShapeDtypeStruct((_N, _D), jnp.float32),
    )(partials, bias2d)


def kernel(x, edge_index, edge_weight, weight, bias):
    support = _matmul(x, weight)
    ew_bits = jax.lax.bitcast_convert_type(edge_weight, jnp.int32)
    eidx3 = jnp.stack(
        [edge_index[0].reshape(_NCHUNKS, _CHUNK),
         edge_index[1].reshape(_NCHUNKS, _CHUNK),
         ew_bits.reshape(_NCHUNKS, _CHUNK)], axis=1)
    partials = _spmm(eidx3, support)
    return _combine(partials, bias.reshape(1, _D))
